# baseline (speedup 1.0000x reference)
import functools

import jax
import jax.numpy as jnp
from jax import lax
from jax.experimental import pallas as pl
from jax.experimental.pallas import tpu as pltpu
from jax.experimental.pallas import tpu_sc as plsc

N = 10000
NR = 10112
F = 128
NC = 2
NS = 16
NW = NC * NS
DEGW = 128
CH = 128
NB = 2


def _sc_mesh():
    return plsc.VectorSubcoreMesh(core_axis_name="c", subcore_axis_name="s")


def _deg_count(dstp, zeros_deg, ones_deg):
    nch = dstp.shape[1]
    rz = NR // NS

    @functools.partial(
        pl.kernel,
        out_type=jax.ShapeDtypeStruct((NC, NR, DEGW), jnp.float32),
        mesh=_sc_mesh(),
        scratch_types=[
            pltpu.VMEM((nch, CH), jnp.int32),
            pltpu.VMEM((CH, DEGW), jnp.float32),
            pltpu.VMEM_SHARED((NR, DEGW), jnp.float32),
            pltpu.SemaphoreType.DMA((NB,)),
        ],
    )
    def k(dstp_hbm, zeros_hbm, ones_hbm, out_hbm, dst_v, ones_v, acc_sh, sem):
        c = lax.axis_index("c")
        s = lax.axis_index("s")
        wid = c * NS + s
        pltpu.sync_copy(dstp_hbm.at[wid], dst_v)
        pltpu.sync_copy(ones_hbm, ones_v)
        pltpu.sync_copy(zeros_hbm.at[pl.ds(s * rz, rz)], acc_sh.at[pl.ds(s * rz, rz)])
        plsc.subcore_barrier()

        for b in range(NB):
            pltpu.async_copy(ones_v, acc_sh.at[dst_v.at[b]], sem.at[b], add=True)

        def body(t, carry):
            for b in range(NB):
                j = t * NB + b
                pltpu.make_async_copy(ones_v, acc_sh.at[dst_v.at[j]], sem.at[b]).wait()

                @pl.when(j + NB < nch)
                def _():
                    pltpu.async_copy(
                        ones_v, acc_sh.at[dst_v.at[j + NB]], sem.at[b], add=True)
            return carry

        lax.fori_loop(0, nch // NB, body, 0)
        plsc.subcore_barrier()
        pltpu.sync_copy(acc_sh.at[pl.ds(s * rz, rz)], out_hbm.at[c, pl.ds(s * rz, rz)])

    return k(dstp, zeros_deg, ones_deg)


def _aggregate(srcp, dstp, g):
    nch = srcp.shape[1]
    hch = nch // 2
    ri = NR // NS

    @functools.partial(
        pl.kernel,
        out_type=jax.ShapeDtypeStruct((NC, NR, F), jnp.float32),
        mesh=_sc_mesh(),
        scratch_types=[
            pltpu.VMEM((hch, CH), jnp.int32),
            pltpu.VMEM((hch, CH), jnp.int32),
            pltpu.VMEM((NB, CH, F), jnp.float32),
            pltpu.VMEM_SHARED((NR, F), jnp.float32),
            pltpu.SemaphoreType.DMA((NB,)),
            pltpu.SemaphoreType.DMA((NB,)),
        ],
    )
    def k(srcp_hbm, dstp_hbm, g_hbm, out_hbm, src_v, dst_v, bufs, acc_sh, sem_g, sem_s):
        c = lax.axis_index("c")
        s = lax.axis_index("s")
        wid = c * NS + s
        pltpu.sync_copy(g_hbm.at[pl.ds(s * ri, ri)], acc_sh.at[pl.ds(s * ri, ri)])
        plsc.subcore_barrier()

        SG = 2
        SC_ = CH // SG

        def _fire_gathers(j, b):
            for q in range(SG):
                pltpu.async_copy(
                    g_hbm.at[src_v.at[j, pl.ds(q * SC_, SC_)]],
                    bufs.at[b, pl.ds(q * SC_, SC_)], sem_g.at[b])

        def _wait_gathers(j, b):
            for q in range(SG):
                pltpu.make_async_copy(
                    g_hbm.at[src_v.at[j, pl.ds(q * SC_, SC_)]],
                    bufs.at[b, pl.ds(q * SC_, SC_)], sem_g.at[b]).wait()

        def grp(t, carry):
            for b in range(NB):
                j = t * NB + b
                _wait_gathers(j, b)
                pltpu.async_copy(
                    bufs.at[b], acc_sh.at[dst_v.at[j]], sem_s.at[b], add=True).wait()

                @pl.when(j + NB < hch)
                def _():
                    _fire_gathers(j + NB, b)
            return carry

        for h in range(2):
            pltpu.sync_copy(srcp_hbm.at[wid, pl.ds(h * hch, hch)], src_v)
            pltpu.sync_copy(dstp_hbm.at[wid, pl.ds(h * hch, hch)], dst_v)
            for b in range(NB):
                _fire_gathers(b, b)
            lax.fori_loop(0, hch // NB, grp, 0)
        plsc.subcore_barrier()
        pltpu.sync_copy(acc_sh.at[pl.ds(s * ri, ri)], out_hbm.at[c, pl.ds(s * ri, ri)])

    return k(srcp, dstp, g)


_TC_R = 1264


def _tc_prep(degcnt, xp, W):
    def body(deg_ref, x_ref, w_ref, g_ref, dinv_ref):
        dd = deg_ref[0, :, :1] + deg_ref[1, :, :1] + 1.0
        dinv = lax.rsqrt(dd)
        h = jnp.dot(x_ref[...], w_ref[...], preferred_element_type=jnp.float32)
        g_ref[...] = dinv * h
        dinv_ref[...] = jnp.broadcast_to(dinv, (_TC_R, F))

    return pl.pallas_call(
        body,
        grid=(NR // _TC_R,),
        in_specs=[
            pl.BlockSpec((NC, _TC_R, DEGW), lambda i: (0, i, 0)),
            pl.BlockSpec((_TC_R, F), lambda i: (i, 0)),
            pl.BlockSpec((F, F), lambda i: (0, 0)),
        ],
        out_specs=[
            pl.BlockSpec((_TC_R, F), lambda i: (i, 0)),
            pl.BlockSpec((_TC_R, F), lambda i: (i, 0)),
        ],
        out_shape=[
            jax.ShapeDtypeStruct((NR, F), jnp.float32),
            jax.ShapeDtypeStruct((NR, F), jnp.float32),
        ],
    )(degcnt, xp, W)


def _tc_mid(accs, g1, dinv_b, W_l, b1, vn0):
    def body(acc_ref, g_ref, dv_ref, w_ref, b_ref, vn_ref, g2_ref):
        out1 = dv_ref[...] * (acc_ref[0] + acc_ref[1] - g_ref[...])
        out1 = out1 + b_ref[...] + vn_ref[...]
        h2 = jnp.dot(out1, w_ref[...], preferred_element_type=jnp.float32)
        g2_ref[...] = dv_ref[...] * h2

    return pl.pallas_call(
        body,
        grid=(NR // _TC_R,),
        in_specs=[
            pl.BlockSpec((NC, _TC_R, F), lambda i: (0, i, 0)),
            pl.BlockSpec((_TC_R, F), lambda i: (i, 0)),
            pl.BlockSpec((_TC_R, F), lambda i: (i, 0)),
            pl.BlockSpec((F, F), lambda i: (0, 0)),
            pl.BlockSpec((1, F), lambda i: (0, 0)),
            pl.BlockSpec((1, F), lambda i: (0, 0)),
        ],
        out_specs=pl.BlockSpec((_TC_R, F), lambda i: (i, 0)),
        out_shape=jax.ShapeDtypeStruct((NR, F), jnp.float32),
    )(accs, g1, dinv_b, W_l, b1, vn0)


def _tc_final(accs, g2, dinv_b, b2):
    R = 1000

    def body(acc_ref, g_ref, dv_ref, b_ref, o_ref):
        o_ref[...] = dv_ref[...] * (acc_ref[0] + acc_ref[1] - g_ref[...]) + b_ref[...]

    return pl.pallas_call(
        body,
        grid=(N // R,),
        in_specs=[
            pl.BlockSpec((NC, R, F), lambda i: (0, i, 0)),
            pl.BlockSpec((R, F), lambda i: (i, 0)),
            pl.BlockSpec((R, F), lambda i: (i, 0)),
            pl.BlockSpec((1, F), lambda i: (0, 0)),
        ],
        out_specs=pl.BlockSpec((R, F), lambda i: (i, 0)),
        out_shape=jax.ShapeDtypeStruct((N, F), jnp.float32),
    )(accs, g2, dinv_b, b2)


def kernel(x, edge_index, h_blocks, h_levels, h_num,
           W_init, b_init, W_l, b_l, vn_emb, mW1, mb1, mW2, mb2):
    E = edge_index.shape[1]
    ept = E // NW
    nch = -(-ept // CH)
    nch = -(-nch // NB) * NB
    pad = nch * CH - ept

    src = edge_index[0].reshape(NW, ept)
    dst = edge_index[1].reshape(NW, ept)
    srcp = jnp.concatenate(
        [src, jnp.zeros((NW, pad), jnp.int32)], axis=1).reshape(NW, nch, CH)
    dstp = jnp.concatenate(
        [dst, jnp.full((NW, pad), N, jnp.int32)], axis=1).reshape(NW, nch, CH)
    xp = jnp.pad(x, ((0, NR - N), (0, 0)))
    zeros_deg = jnp.zeros((NR, DEGW), jnp.float32)
    ones_deg = jnp.ones((CH, DEGW), jnp.float32)

    degcnt = _deg_count(dstp, zeros_deg, ones_deg)
    g1, dinv_b = _tc_prep(degcnt, xp, W_init)
    accs1 = _aggregate(srcp, dstp, g1)
    g2 = _tc_mid(accs1, g1, dinv_b, W_l, b_init.reshape(1, F), vn_emb)
    accs2 = _aggregate(srcp, dstp, g2)
    return _tc_final(accs2, g2, dinv_b, b_l.reshape(1, F))


# speedup vs baseline: 1.0020x; 1.0020x over previous
import functools

import jax
import jax.numpy as jnp
from jax import lax
from jax.experimental import pallas as pl
from jax.experimental.pallas import tpu as pltpu
from jax.experimental.pallas import tpu_sc as plsc

N = 10000
NR = 10112
F = 128
NC = 2
NS = 16
NW = NC * NS
DEGW = 128
CH = 128
NB = 2


def _sc_mesh():
    return plsc.VectorSubcoreMesh(core_axis_name="c", subcore_axis_name="s")


def _deg_count(dstp, zeros_deg, ones_deg):
    nch = dstp.shape[1]
    rz = NR // NS

    @functools.partial(
        pl.kernel,
        out_type=jax.ShapeDtypeStruct((NC, NR, DEGW), jnp.float32),
        mesh=_sc_mesh(),
        scratch_types=[
            pltpu.VMEM((nch, CH), jnp.int32),
            pltpu.VMEM((CH, DEGW), jnp.float32),
            pltpu.VMEM_SHARED((NR, DEGW), jnp.float32),
            pltpu.SemaphoreType.DMA((4,)),
        ],
    )
    def k(dstp_hbm, zeros_hbm, ones_hbm, out_hbm, dst_v, ones_v, acc_sh, sem):
        c = lax.axis_index("c")
        s = lax.axis_index("s")
        wid = c * NS + s
        pltpu.sync_copy(dstp_hbm.at[wid], dst_v)
        pltpu.sync_copy(ones_hbm, ones_v)
        pltpu.sync_copy(zeros_hbm.at[pl.ds(s * rz, rz)], acc_sh.at[pl.ds(s * rz, rz)])
        plsc.subcore_barrier()

        ND = 4
        for b in range(ND):
            pltpu.async_copy(ones_v, acc_sh.at[dst_v.at[b]], sem.at[b], add=True)

        def body(t, carry):
            for b in range(ND):
                j = t * ND + b
                pltpu.make_async_copy(ones_v, acc_sh.at[dst_v.at[j]], sem.at[b]).wait()

                @pl.when(j + ND < nch)
                def _():
                    pltpu.async_copy(
                        ones_v, acc_sh.at[dst_v.at[j + ND]], sem.at[b], add=True)
            return carry

        lax.fori_loop(0, nch // ND, body, 0)
        plsc.subcore_barrier()
        pltpu.sync_copy(acc_sh.at[pl.ds(s * rz, rz)], out_hbm.at[c, pl.ds(s * rz, rz)])

    return k(dstp, zeros_deg, ones_deg)


def _aggregate(srcp, dstp, g):
    nch = srcp.shape[1]
    hch = nch // 2
    ri = NR // NS

    @functools.partial(
        pl.kernel,
        out_type=jax.ShapeDtypeStruct((NC, NR, F), jnp.float32),
        mesh=_sc_mesh(),
        scratch_types=[
            pltpu.VMEM((hch, CH), jnp.int32),
            pltpu.VMEM((hch, CH), jnp.int32),
            pltpu.VMEM((NB, CH, F), jnp.float32),
            pltpu.VMEM_SHARED((NR, F), jnp.float32),
            pltpu.SemaphoreType.DMA((NB,)),
            pltpu.SemaphoreType.DMA((NB,)),
        ],
    )
    def k(srcp_hbm, dstp_hbm, g_hbm, out_hbm, src_v, dst_v, bufs, acc_sh, sem_g, sem_s):
        c = lax.axis_index("c")
        s = lax.axis_index("s")
        wid = c * NS + s
        pltpu.sync_copy(g_hbm.at[pl.ds(s * ri, ri)], acc_sh.at[pl.ds(s * ri, ri)])
        plsc.subcore_barrier()

        def grp(t, carry):
            for b in range(NB):
                j = t * NB + b
                pltpu.make_async_copy(
                    g_hbm.at[src_v.at[j]], bufs.at[b], sem_g.at[b]).wait()
                pltpu.async_copy(
                    bufs.at[b], acc_sh.at[dst_v.at[j]], sem_s.at[b], add=True).wait()

                @pl.when(j + NB < hch)
                def _():
                    pltpu.async_copy(
                        g_hbm.at[src_v.at[j + NB]], bufs.at[b], sem_g.at[b])
            return carry

        for h in range(2):
            pltpu.sync_copy(srcp_hbm.at[wid, pl.ds(h * hch, hch)], src_v)
            pltpu.sync_copy(dstp_hbm.at[wid, pl.ds(h * hch, hch)], dst_v)
            for b in range(NB):
                pltpu.async_copy(g_hbm.at[src_v.at[b]], bufs.at[b], sem_g.at[b])
            lax.fori_loop(0, hch // NB, grp, 0)
        plsc.subcore_barrier()
        pltpu.sync_copy(acc_sh.at[pl.ds(s * ri, ri)], out_hbm.at[c, pl.ds(s * ri, ri)])

    return k(srcp, dstp, g)


_TC_R = 1264


def _tc_matmul(xp, W):
    def body(x_ref, w_ref, h_ref):
        h_ref[...] = jnp.dot(x_ref[...], w_ref[...], preferred_element_type=jnp.float32)

    return pl.pallas_call(
        body,
        grid=(NR // _TC_R,),
        in_specs=[
            pl.BlockSpec((_TC_R, F), lambda i: (i, 0)),
            pl.BlockSpec((F, F), lambda i: (0, 0)),
        ],
        out_specs=pl.BlockSpec((_TC_R, F), lambda i: (i, 0)),
        out_shape=jax.ShapeDtypeStruct((NR, F), jnp.float32),
    )(xp, W)


def _tc_scale(degcnt, h1):
    def body(deg_ref, h_ref, g_ref, dinv_ref):
        dd = deg_ref[0, :, :1] + deg_ref[1, :, :1] + 1.0
        dinv = lax.rsqrt(dd)
        g_ref[...] = dinv * h_ref[...]
        dinv_ref[...] = jnp.broadcast_to(dinv, (_TC_R, F))

    return pl.pallas_call(
        body,
        grid=(NR // _TC_R,),
        in_specs=[
            pl.BlockSpec((NC, _TC_R, DEGW), lambda i: (0, i, 0)),
            pl.BlockSpec((_TC_R, F), lambda i: (i, 0)),
        ],
        out_specs=[
            pl.BlockSpec((_TC_R, F), lambda i: (i, 0)),
            pl.BlockSpec((_TC_R, F), lambda i: (i, 0)),
        ],
        out_shape=[
            jax.ShapeDtypeStruct((NR, F), jnp.float32),
            jax.ShapeDtypeStruct((NR, F), jnp.float32),
        ],
    )(degcnt, h1)


def _tc_mid(accs, g1, dinv_b, W_l, b1, vn0):
    def body(acc_ref, g_ref, dv_ref, w_ref, b_ref, vn_ref, g2_ref):
        out1 = dv_ref[...] * (acc_ref[0] + acc_ref[1] - g_ref[...])
        out1 = out1 + b_ref[...] + vn_ref[...]
        h2 = jnp.dot(out1, w_ref[...], preferred_element_type=jnp.float32)
        g2_ref[...] = dv_ref[...] * h2

    return pl.pallas_call(
        body,
        grid=(NR // _TC_R,),
        in_specs=[
            pl.BlockSpec((NC, _TC_R, F), lambda i: (0, i, 0)),
            pl.BlockSpec((_TC_R, F), lambda i: (i, 0)),
            pl.BlockSpec((_TC_R, F), lambda i: (i, 0)),
            pl.BlockSpec((F, F), lambda i: (0, 0)),
            pl.BlockSpec((1, F), lambda i: (0, 0)),
            pl.BlockSpec((1, F), lambda i: (0, 0)),
        ],
        out_specs=pl.BlockSpec((_TC_R, F), lambda i: (i, 0)),
        out_shape=jax.ShapeDtypeStruct((NR, F), jnp.float32),
    )(accs, g1, dinv_b, W_l, b1, vn0)


def _tc_final(accs, g2, dinv_b, b2):
    R = 1000

    def body(acc_ref, g_ref, dv_ref, b_ref, o_ref):
        o_ref[...] = dv_ref[...] * (acc_ref[0] + acc_ref[1] - g_ref[...]) + b_ref[...]

    return pl.pallas_call(
        body,
        grid=(N // R,),
        in_specs=[
            pl.BlockSpec((NC, R, F), lambda i: (0, i, 0)),
            pl.BlockSpec((R, F), lambda i: (i, 0)),
            pl.BlockSpec((R, F), lambda i: (i, 0)),
            pl.BlockSpec((1, F), lambda i: (0, 0)),
        ],
        out_specs=pl.BlockSpec((R, F), lambda i: (i, 0)),
        out_shape=jax.ShapeDtypeStruct((N, F), jnp.float32),
    )(accs, g2, dinv_b, b2)


def kernel(x, edge_index, h_blocks, h_levels, h_num,
           W_init, b_init, W_l, b_l, vn_emb, mW1, mb1, mW2, mb2):
    E = edge_index.shape[1]
    ept = E // NW
    nch = -(-ept // CH)
    nch = -(-nch // NB) * NB
    pad = nch * CH - ept

    src = edge_index[0].reshape(NW, ept)
    dst = edge_index[1].reshape(NW, ept)
    srcp = jnp.concatenate(
        [src, jnp.zeros((NW, pad), jnp.int32)], axis=1).reshape(NW, nch, CH)
    dstp = jnp.concatenate(
        [dst, jnp.full((NW, pad), N, jnp.int32)], axis=1).reshape(NW, nch, CH)
    xp = jnp.pad(x, ((0, NR - N), (0, 0)))
    zeros_deg = jnp.zeros((NR, DEGW), jnp.float32)
    ones_deg = jnp.ones((CH, DEGW), jnp.float32)

    degcnt = _deg_count(dstp, zeros_deg, ones_deg)
    h1 = _tc_matmul(xp, W_init)
    g1, dinv_b = _tc_scale(degcnt, h1)
    accs1 = _aggregate(srcp, dstp, g1)
    g2 = _tc_mid(accs1, g1, dinv_b, W_l, b_init.reshape(1, F), vn_emb)
    accs2 = _aggregate(srcp, dstp, g2)
    return _tc_final(accs2, g2, dinv_b, b_l.reshape(1, F))


# speedup vs baseline: 2.5513x; 2.5462x over previous
import functools

import jax
import jax.numpy as jnp
from jax import lax
from jax.experimental import pallas as pl
from jax.experimental.pallas import tpu as pltpu
from jax.experimental.pallas import tpu_sc as plsc

N = 10000
NR = 10112
F = 128
NC = 2
NS = 16
NW = NC * NS
DEGW = 128
CH = 128
NB = 2


def _sc_mesh():
    return plsc.VectorSubcoreMesh(core_axis_name="c", subcore_axis_name="s")


def _deg_count(dstp, zeros_deg, ones_deg):
    nch = dstp.shape[1]
    rz = NR // NS

    @functools.partial(
        pl.kernel,
        out_type=jax.ShapeDtypeStruct((NC, NR, DEGW), jnp.float32),
        mesh=_sc_mesh(),
        scratch_types=[
            pltpu.VMEM((nch, CH), jnp.int32),
            pltpu.VMEM((CH, DEGW), jnp.float32),
            pltpu.VMEM_SHARED((NR, DEGW), jnp.float32),
            pltpu.SemaphoreType.DMA((4,)),
        ],
    )
    def k(dstp_hbm, zeros_hbm, ones_hbm, out_hbm, dst_v, ones_v, acc_sh, sem):
        c = lax.axis_index("c")
        s = lax.axis_index("s")
        wid = c * NS + s
        pltpu.sync_copy(dstp_hbm.at[wid], dst_v)
        pltpu.sync_copy(ones_hbm, ones_v)
        pltpu.sync_copy(zeros_hbm.at[pl.ds(s * rz, rz)], acc_sh.at[pl.ds(s * rz, rz)])
        plsc.subcore_barrier()

        ND = 4
        for b in range(ND):
            pltpu.async_copy(ones_v, acc_sh.at[dst_v.at[b]], sem.at[b], add=True)

        def body(t, carry):
            for b in range(ND):
                j = t * ND + b
                pltpu.make_async_copy(ones_v, acc_sh.at[dst_v.at[j]], sem.at[b]).wait()

                @pl.when(j + ND < nch)
                def _():
                    pltpu.async_copy(
                        ones_v, acc_sh.at[dst_v.at[j + ND]], sem.at[b], add=True)
            return carry

        lax.fori_loop(0, nch // ND, body, 0)
        plsc.subcore_barrier()
        pltpu.sync_copy(acc_sh.at[pl.ds(s * rz, rz)], out_hbm.at[c, pl.ds(s * rz, rz)])

    return k(dstp, zeros_deg, ones_deg)


def _aggregate(srcp, dstp, g):
    nch = srcp.shape[1]
    hch = nch // 2
    ri = NR // NS

    @functools.partial(
        pl.kernel,
        out_type=jax.ShapeDtypeStruct((NC, NR, F), jnp.float32),
        mesh=_sc_mesh(),
        scratch_types=[
            pltpu.VMEM((hch, CH), jnp.int32),
            pltpu.VMEM((hch, CH), jnp.int32),
            pltpu.VMEM((NB, CH, F), jnp.float32),
            pltpu.VMEM_SHARED((NR, F), jnp.float32),
            pltpu.SemaphoreType.DMA((NB,)),
            pltpu.SemaphoreType.DMA((NB,)),
        ],
    )
    def k(srcp_hbm, dstp_hbm, g_hbm, out_hbm, src_v, dst_v, bufs, acc_sh, sem_g, sem_s):
        c = lax.axis_index("c")
        s = lax.axis_index("s")
        wid = c * NS + s
        pltpu.sync_copy(g_hbm.at[pl.ds(s * ri, ri)], acc_sh.at[pl.ds(s * ri, ri)])
        plsc.subcore_barrier()

        def grp(t, carry):
            for b in range(NB):
                j = t * NB + b
                pltpu.make_async_copy(
                    g_hbm.at[src_v.at[j]], bufs.at[b], sem_g.at[b]).wait()
                pltpu.async_copy(
                    bufs.at[b], acc_sh.at[dst_v.at[j]], sem_s.at[b], add=True).wait()

                @pl.when(j + NB < hch)
                def _():
                    pltpu.async_copy(
                        g_hbm.at[src_v.at[j + NB]], bufs.at[b], sem_g.at[b])
            return carry

        for h in range(2):
            pltpu.sync_copy(srcp_hbm.at[wid, pl.ds(h * hch, hch)], src_v)
            pltpu.sync_copy(dstp_hbm.at[wid, pl.ds(h * hch, hch)], dst_v)
            for b in range(NB):
                pltpu.async_copy(g_hbm.at[src_v.at[b]], bufs.at[b], sem_g.at[b])
            lax.fori_loop(0, hch // NB, grp, 0)
        plsc.subcore_barrier()
        pltpu.sync_copy(acc_sh.at[pl.ds(s * ri, ri)], out_hbm.at[c, pl.ds(s * ri, ri)])

    return k(srcp, dstp, g)


_TC_R = 1264


def _tc_matmul(xp, W):
    def body(x_ref, w_ref, h_ref):
        h_ref[...] = jnp.dot(x_ref[...], w_ref[...], preferred_element_type=jnp.float32)

    return pl.pallas_call(
        body,
        grid=(NR // _TC_R,),
        in_specs=[
            pl.BlockSpec((_TC_R, F), lambda i: (i, 0)),
            pl.BlockSpec((F, F), lambda i: (0, 0)),
        ],
        out_specs=pl.BlockSpec((_TC_R, F), lambda i: (i, 0)),
        out_shape=jax.ShapeDtypeStruct((NR, F), jnp.float32),
    )(xp, W)


def _tc_scale(degcnt, h1):
    def body(deg_ref, h_ref, g_ref, dinv_ref):
        dd = deg_ref[0, :, :1] + deg_ref[1, :, :1] + 1.0
        dinv = lax.rsqrt(dd)
        g_ref[...] = dinv * h_ref[...]
        dinv_ref[...] = jnp.broadcast_to(dinv, (_TC_R, F))

    return pl.pallas_call(
        body,
        grid=(NR // _TC_R,),
        in_specs=[
            pl.BlockSpec((NC, _TC_R, DEGW), lambda i: (0, i, 0)),
            pl.BlockSpec((_TC_R, F), lambda i: (i, 0)),
        ],
        out_specs=[
            pl.BlockSpec((_TC_R, F), lambda i: (i, 0)),
            pl.BlockSpec((_TC_R, F), lambda i: (i, 0)),
        ],
        out_shape=[
            jax.ShapeDtypeStruct((NR, F), jnp.float32),
            jax.ShapeDtypeStruct((NR, F), jnp.float32),
        ],
    )(degcnt, h1)


def _tc_mid(accs, g1, dinv_b, W_l, b1, vn0):
    def body(acc_ref, g_ref, dv_ref, w_ref, b_ref, vn_ref, g2_ref):
        out1 = dv_ref[...] * (acc_ref[0] + acc_ref[1] - g_ref[...])
        out1 = out1 + b_ref[...] + vn_ref[...]
        h2 = jnp.dot(out1, w_ref[...], preferred_element_type=jnp.float32)
        g2_ref[...] = dv_ref[...] * h2

    return pl.pallas_call(
        body,
        grid=(NR // _TC_R,),
        in_specs=[
            pl.BlockSpec((NC, _TC_R, F), lambda i: (0, i, 0)),
            pl.BlockSpec((_TC_R, F), lambda i: (i, 0)),
            pl.BlockSpec((_TC_R, F), lambda i: (i, 0)),
            pl.BlockSpec((F, F), lambda i: (0, 0)),
            pl.BlockSpec((1, F), lambda i: (0, 0)),
            pl.BlockSpec((1, F), lambda i: (0, 0)),
        ],
        out_specs=pl.BlockSpec((_TC_R, F), lambda i: (i, 0)),
        out_shape=jax.ShapeDtypeStruct((NR, F), jnp.float32),
    )(accs, g1, dinv_b, W_l, b1, vn0)


def _tc_final(accs, g2, dinv_b, b2):
    R = 1000

    def body(acc_ref, g_ref, dv_ref, b_ref, o_ref):
        o_ref[...] = dv_ref[...] * (acc_ref[0] + acc_ref[1] - g_ref[...]) + b_ref[...]

    return pl.pallas_call(
        body,
        grid=(N // R,),
        in_specs=[
            pl.BlockSpec((NC, R, F), lambda i: (0, i, 0)),
            pl.BlockSpec((R, F), lambda i: (i, 0)),
            pl.BlockSpec((R, F), lambda i: (i, 0)),
            pl.BlockSpec((1, F), lambda i: (0, 0)),
        ],
        out_specs=pl.BlockSpec((R, F), lambda i: (i, 0)),
        out_shape=jax.ShapeDtypeStruct((N, F), jnp.float32),
    )(accs, g2, dinv_b, b2)


def kernel(x, edge_index, h_blocks, h_levels, h_num,
           W_init, b_init, W_l, b_l, vn_emb, mW1, mb1, mW2, mb2):
    E = edge_index.shape[1]
    ept = E // NW
    nch = -(-ept // CH)
    nch = -(-nch // NB) * NB
    pad = nch * CH - ept

    src = edge_index[0].reshape(NW, ept)
    dst = edge_index[1].reshape(NW, ept)
    spread = jnp.arange(pad, dtype=jnp.int32)
    pad_src = jnp.broadcast_to(spread * 79 % N, (NW, pad))
    pad_dst = jnp.broadcast_to(N + spread % (NR - N), (NW, pad))
    srcp = jnp.concatenate([src, pad_src], axis=1).reshape(NW, nch, CH)
    dstp = jnp.concatenate([dst, pad_dst], axis=1).reshape(NW, nch, CH)
    xp = jnp.pad(x, ((0, NR - N), (0, 0)))
    zeros_deg = jnp.zeros((NR, DEGW), jnp.float32)
    ones_deg = jnp.ones((CH, DEGW), jnp.float32)

    degcnt = _deg_count(dstp, zeros_deg, ones_deg)
    h1 = _tc_matmul(xp, W_init)
    g1, dinv_b = _tc_scale(degcnt, h1)
    accs1 = _aggregate(srcp, dstp, g1)
    g2 = _tc_mid(accs1, g1, dinv_b, W_l, b_init.reshape(1, F), vn_emb)
    accs2 = _aggregate(srcp, dstp, g2)
    return _tc_final(accs2, g2, dinv_b, b_l.reshape(1, F))


# speedup vs baseline: 2.5520x; 1.0003x over previous
import functools

import jax
import jax.numpy as jnp
from jax import lax
from jax.experimental import pallas as pl
from jax.experimental.pallas import tpu as pltpu
from jax.experimental.pallas import tpu_sc as plsc

N = 10000
NR = 10112
F = 128
NC = 2
NS = 16
NW = NC * NS
DEGW = 128
CH = 128
NB = 2


def _sc_mesh():
    return plsc.VectorSubcoreMesh(core_axis_name="c", subcore_axis_name="s")


def _deg_count(dstp, zeros_deg, ones_deg):
    nch = dstp.shape[1]
    rz = NR // NS

    @functools.partial(
        pl.kernel,
        out_type=jax.ShapeDtypeStruct((NC, NR, DEGW), jnp.float32),
        mesh=_sc_mesh(),
        scratch_types=[
            pltpu.VMEM((nch, CH), jnp.int32),
            pltpu.VMEM((CH, DEGW), jnp.float32),
            pltpu.VMEM_SHARED((NR, DEGW), jnp.float32),
            pltpu.SemaphoreType.DMA((4,)),
        ],
    )
    def k(dstp_hbm, zeros_hbm, ones_hbm, out_hbm, dst_v, ones_v, acc_sh, sem):
        c = lax.axis_index("c")
        s = lax.axis_index("s")
        wid = c * NS + s
        pltpu.sync_copy(dstp_hbm.at[wid], dst_v)
        pltpu.sync_copy(ones_hbm, ones_v)
        pltpu.sync_copy(zeros_hbm.at[pl.ds(s * rz, rz)], acc_sh.at[pl.ds(s * rz, rz)])
        plsc.subcore_barrier()

        ND = 4
        for b in range(ND):
            pltpu.async_copy(ones_v, acc_sh.at[dst_v.at[b]], sem.at[b], add=True)

        def body(t, carry):
            for b in range(ND):
                j = t * ND + b
                pltpu.make_async_copy(ones_v, acc_sh.at[dst_v.at[j]], sem.at[b]).wait()

                @pl.when(j + ND < nch)
                def _():
                    pltpu.async_copy(
                        ones_v, acc_sh.at[dst_v.at[j + ND]], sem.at[b], add=True)
            return carry

        lax.fori_loop(0, nch // ND, body, 0)
        plsc.subcore_barrier()
        pltpu.sync_copy(acc_sh.at[pl.ds(s * rz, rz)], out_hbm.at[c, pl.ds(s * rz, rz)])

    return k(dstp, zeros_deg, ones_deg)


def _aggregate(srcp, dstp, g):
    nch = srcp.shape[1]
    hch = nch // 2
    ri = NR // NS

    @functools.partial(
        pl.kernel,
        out_type=jax.ShapeDtypeStruct((NC, NR, F), jnp.float32),
        mesh=_sc_mesh(),
        scratch_types=[
            pltpu.VMEM((hch, CH), jnp.int32),
            pltpu.VMEM((hch, CH), jnp.int32),
            pltpu.VMEM((NB, CH, F), jnp.float32),
            pltpu.VMEM_SHARED((NR, F), jnp.float32),
            pltpu.SemaphoreType.DMA((NB,)),
            pltpu.SemaphoreType.DMA((NB,)),
        ],
    )
    def k(srcp_hbm, dstp_hbm, g_hbm, out_hbm, src_v, dst_v, bufs, acc_sh, sem_g, sem_s):
        c = lax.axis_index("c")
        s = lax.axis_index("s")
        wid = c * NS + s
        pltpu.sync_copy(g_hbm.at[pl.ds(s * ri, ri)], acc_sh.at[pl.ds(s * ri, ri)])
        plsc.subcore_barrier()

        def grp(t, carry):
            for b in range(NB):
                j = t * NB + b
                pltpu.make_async_copy(
                    g_hbm.at[src_v.at[j]], bufs.at[b], sem_g.at[b]).wait()
                pltpu.async_copy(
                    bufs.at[b], acc_sh.at[dst_v.at[j]], sem_s.at[b], add=True).wait()

                @pl.when(j + NB < hch)
                def _():
                    pltpu.async_copy(
                        g_hbm.at[src_v.at[j + NB]], bufs.at[b], sem_g.at[b])
            return carry

        for h in range(2):
            pltpu.sync_copy(srcp_hbm.at[wid, pl.ds(h * hch, hch)], src_v)
            pltpu.sync_copy(dstp_hbm.at[wid, pl.ds(h * hch, hch)], dst_v)
            for b in range(NB):
                pltpu.async_copy(g_hbm.at[src_v.at[b]], bufs.at[b], sem_g.at[b])
            lax.fori_loop(0, hch // NB, grp, 0)
        plsc.subcore_barrier()
        pltpu.sync_copy(acc_sh.at[pl.ds(s * ri, ri)], out_hbm.at[c, pl.ds(s * ri, ri)])

    return k(srcp, dstp, g)


_TC_R = 1264


def _tc_prep(degcnt, xp, W):
    def body(deg_ref, x_ref, w_ref, g_ref, dinv_ref):
        dd = deg_ref[0, :, :1] + deg_ref[1, :, :1] + 1.0
        dinv = lax.rsqrt(dd)
        h = jnp.dot(x_ref[...], w_ref[...], preferred_element_type=jnp.float32)
        g_ref[...] = dinv * h
        dinv_ref[...] = jnp.broadcast_to(dinv, (_TC_R, F))

    return pl.pallas_call(
        body,
        grid=(NR // _TC_R,),
        in_specs=[
            pl.BlockSpec((NC, _TC_R, DEGW), lambda i: (0, i, 0)),
            pl.BlockSpec((_TC_R, F), lambda i: (i, 0)),
            pl.BlockSpec((F, F), lambda i: (0, 0)),
        ],
        out_specs=[
            pl.BlockSpec((_TC_R, F), lambda i: (i, 0)),
            pl.BlockSpec((_TC_R, F), lambda i: (i, 0)),
        ],
        out_shape=[
            jax.ShapeDtypeStruct((NR, F), jnp.float32),
            jax.ShapeDtypeStruct((NR, F), jnp.float32),
        ],
    )(degcnt, xp, W)


def _tc_mid(accs, g1, dinv_b, W_l, b1, vn0):
    def body(acc_ref, g_ref, dv_ref, w_ref, b_ref, vn_ref, g2_ref):
        out1 = dv_ref[...] * (acc_ref[0] + acc_ref[1] - g_ref[...])
        out1 = out1 + b_ref[...] + vn_ref[...]
        h2 = jnp.dot(out1, w_ref[...], preferred_element_type=jnp.float32)
        g2_ref[...] = dv_ref[...] * h2

    return pl.pallas_call(
        body,
        grid=(NR // _TC_R,),
        in_specs=[
            pl.BlockSpec((NC, _TC_R, F), lambda i: (0, i, 0)),
            pl.BlockSpec((_TC_R, F), lambda i: (i, 0)),
            pl.BlockSpec((_TC_R, F), lambda i: (i, 0)),
            pl.BlockSpec((F, F), lambda i: (0, 0)),
            pl.BlockSpec((1, F), lambda i: (0, 0)),
            pl.BlockSpec((1, F), lambda i: (0, 0)),
        ],
        out_specs=pl.BlockSpec((_TC_R, F), lambda i: (i, 0)),
        out_shape=jax.ShapeDtypeStruct((NR, F), jnp.float32),
    )(accs, g1, dinv_b, W_l, b1, vn0)


def _tc_final(accs, g2, dinv_b, b2):
    R = 1000

    def body(acc_ref, g_ref, dv_ref, b_ref, o_ref):
        o_ref[...] = dv_ref[...] * (acc_ref[0] + acc_ref[1] - g_ref[...]) + b_ref[...]

    return pl.pallas_call(
        body,
        grid=(N // R,),
        in_specs=[
            pl.BlockSpec((NC, R, F), lambda i: (0, i, 0)),
            pl.BlockSpec((R, F), lambda i: (i, 0)),
            pl.BlockSpec((R, F), lambda i: (i, 0)),
            pl.BlockSpec((1, F), lambda i: (0, 0)),
        ],
        out_specs=pl.BlockSpec((R, F), lambda i: (i, 0)),
        out_shape=jax.ShapeDtypeStruct((N, F), jnp.float32),
    )(accs, g2, dinv_b, b2)


def kernel(x, edge_index, h_blocks, h_levels, h_num,
           W_init, b_init, W_l, b_l, vn_emb, mW1, mb1, mW2, mb2):
    E = edge_index.shape[1]
    ept = E // NW
    nch = -(-ept // CH)
    nch = -(-nch // NB) * NB
    pad = nch * CH - ept

    src = edge_index[0].reshape(NW, ept)
    dst = edge_index[1].reshape(NW, ept)
    spread = jnp.arange(pad, dtype=jnp.int32)
    pad_src = jnp.broadcast_to(spread * 79 % N, (NW, pad))
    pad_dst = jnp.broadcast_to(N + spread % (NR - N), (NW, pad))
    srcp = jnp.concatenate([src, pad_src], axis=1).reshape(NW, nch, CH)
    dstp = jnp.concatenate([dst, pad_dst], axis=1).reshape(NW, nch, CH)
    xp = jnp.pad(x, ((0, NR - N), (0, 0)))
    zeros_deg = jnp.zeros((NR, DEGW), jnp.float32)
    ones_deg = jnp.ones((CH, DEGW), jnp.float32)

    degcnt = _deg_count(dstp, zeros_deg, ones_deg)
    g1, dinv_b = _tc_prep(degcnt, xp, W_init)
    accs1 = _aggregate(srcp, dstp, g1)
    g2 = _tc_mid(accs1, g1, dinv_b, W_l, b_init.reshape(1, F), vn_emb)
    accs2 = _aggregate(srcp, dstp, g2)
    return _tc_final(accs2, g2, dinv_b, b_l.reshape(1, F))


# speedup vs baseline: 2.5575x; 1.0022x over previous
import functools

import jax
import jax.numpy as jnp
from jax import lax
from jax.experimental import pallas as pl
from jax.experimental.pallas import tpu as pltpu
from jax.experimental.pallas import tpu_sc as plsc

N = 10000
NR = 10112
F = 128
NC = 2
NS = 16
NW = NC * NS
DEGW = 128
CH = 128
NB = 2


def _sc_mesh():
    return plsc.VectorSubcoreMesh(core_axis_name="c", subcore_axis_name="s")


def _deg_count(dstp, zeros_deg, ones_deg):
    nch = dstp.shape[1]
    rz = NR // NS

    @functools.partial(
        pl.kernel,
        out_type=jax.ShapeDtypeStruct((NC, NR, DEGW), jnp.float32),
        mesh=_sc_mesh(),
        scratch_types=[
            pltpu.VMEM((nch, CH), jnp.int32),
            pltpu.VMEM((CH, DEGW), jnp.float32),
            pltpu.VMEM_SHARED((NR, DEGW), jnp.float32),
            pltpu.SemaphoreType.DMA((8,)),
        ],
    )
    def k(dstp_hbm, zeros_hbm, ones_hbm, out_hbm, dst_v, ones_v, acc_sh, sem):
        c = lax.axis_index("c")
        s = lax.axis_index("s")
        wid = c * NS + s
        pltpu.sync_copy(dstp_hbm.at[wid], dst_v)
        pltpu.sync_copy(ones_hbm, ones_v)
        pltpu.sync_copy(zeros_hbm.at[pl.ds(s * rz, rz)], acc_sh.at[pl.ds(s * rz, rz)])
        plsc.subcore_barrier()

        ND = 8
        for b in range(ND):
            pltpu.async_copy(ones_v, acc_sh.at[dst_v.at[b]], sem.at[b], add=True)

        def body(t, carry):
            for b in range(ND):
                j = t * ND + b
                pltpu.make_async_copy(ones_v, acc_sh.at[dst_v.at[j]], sem.at[b]).wait()

                @pl.when(j + ND < nch)
                def _():
                    pltpu.async_copy(
                        ones_v, acc_sh.at[dst_v.at[j + ND]], sem.at[b], add=True)
            return carry

        lax.fori_loop(0, nch // ND, body, 0)
        plsc.subcore_barrier()
        pltpu.sync_copy(acc_sh.at[pl.ds(s * rz, rz)], out_hbm.at[c, pl.ds(s * rz, rz)])

    return k(dstp, zeros_deg, ones_deg)


def _aggregate(srcp, dstp, g):
    nch = srcp.shape[1]
    hch = nch // 2
    ri = NR // NS

    @functools.partial(
        pl.kernel,
        out_type=jax.ShapeDtypeStruct((NC, NR, F), jnp.float32),
        mesh=_sc_mesh(),
        scratch_types=[
            pltpu.VMEM((hch, CH), jnp.int32),
            pltpu.VMEM((hch, CH), jnp.int32),
            pltpu.VMEM((NB, CH, F), jnp.float32),
            pltpu.VMEM_SHARED((NR, F), jnp.float32),
            pltpu.SemaphoreType.DMA((NB,)),
            pltpu.SemaphoreType.DMA((NB,)),
        ],
    )
    def k(srcp_hbm, dstp_hbm, g_hbm, out_hbm, src_v, dst_v, bufs, acc_sh, sem_g, sem_s):
        c = lax.axis_index("c")
        s = lax.axis_index("s")
        wid = c * NS + s
        pltpu.sync_copy(g_hbm.at[pl.ds(s * ri, ri)], acc_sh.at[pl.ds(s * ri, ri)])
        plsc.subcore_barrier()

        def grp(t, carry):
            for b in range(NB):
                j = t * NB + b
                pltpu.make_async_copy(
                    g_hbm.at[src_v.at[j]], bufs.at[b], sem_g.at[b]).wait()
                pltpu.async_copy(
                    bufs.at[b], acc_sh.at[dst_v.at[j]], sem_s.at[b], add=True).wait()

                @pl.when(j + NB < hch)
                def _():
                    pltpu.async_copy(
                        g_hbm.at[src_v.at[j + NB]], bufs.at[b], sem_g.at[b])
            return carry

        for h in range(2):
            pltpu.sync_copy(srcp_hbm.at[wid, pl.ds(h * hch, hch)], src_v)
            pltpu.sync_copy(dstp_hbm.at[wid, pl.ds(h * hch, hch)], dst_v)
            for b in range(NB):
                pltpu.async_copy(g_hbm.at[src_v.at[b]], bufs.at[b], sem_g.at[b])
            lax.fori_loop(0, hch // NB, grp, 0)
        plsc.subcore_barrier()
        pltpu.sync_copy(acc_sh.at[pl.ds(s * ri, ri)], out_hbm.at[c, pl.ds(s * ri, ri)])

    return k(srcp, dstp, g)


_TC_R = 1264


def _tc_prep(degcnt, xp, W):
    def body(deg_ref, x_ref, w_ref, g_ref, dinv_ref):
        dd = deg_ref[0, :, :1] + deg_ref[1, :, :1] + 1.0
        dinv = lax.rsqrt(dd)
        h = jnp.dot(x_ref[...], w_ref[...], preferred_element_type=jnp.float32)
        g_ref[...] = dinv * h
        dinv_ref[...] = jnp.broadcast_to(dinv, (_TC_R, F))

    return pl.pallas_call(
        body,
        grid=(NR // _TC_R,),
        in_specs=[
            pl.BlockSpec((NC, _TC_R, DEGW), lambda i: (0, i, 0)),
            pl.BlockSpec((_TC_R, F), lambda i: (i, 0)),
            pl.BlockSpec((F, F), lambda i: (0, 0)),
        ],
        out_specs=[
            pl.BlockSpec((_TC_R, F), lambda i: (i, 0)),
            pl.BlockSpec((_TC_R, F), lambda i: (i, 0)),
        ],
        out_shape=[
            jax.ShapeDtypeStruct((NR, F), jnp.float32),
            jax.ShapeDtypeStruct((NR, F), jnp.float32),
        ],
    )(degcnt, xp, W)


def _tc_mid(accs, g1, dinv_b, W_l, b1, vn0):
    def body(acc_ref, g_ref, dv_ref, w_ref, b_ref, vn_ref, g2_ref):
        out1 = dv_ref[...] * (acc_ref[0] + acc_ref[1] - g_ref[...])
        out1 = out1 + b_ref[...] + vn_ref[...]
        h2 = jnp.dot(out1, w_ref[...], preferred_element_type=jnp.float32)
        g2_ref[...] = dv_ref[...] * h2

    return pl.pallas_call(
        body,
        grid=(NR // _TC_R,),
        in_specs=[
            pl.BlockSpec((NC, _TC_R, F), lambda i: (0, i, 0)),
            pl.BlockSpec((_TC_R, F), lambda i: (i, 0)),
            pl.BlockSpec((_TC_R, F), lambda i: (i, 0)),
            pl.BlockSpec((F, F), lambda i: (0, 0)),
            pl.BlockSpec((1, F), lambda i: (0, 0)),
            pl.BlockSpec((1, F), lambda i: (0, 0)),
        ],
        out_specs=pl.BlockSpec((_TC_R, F), lambda i: (i, 0)),
        out_shape=jax.ShapeDtypeStruct((NR, F), jnp.float32),
    )(accs, g1, dinv_b, W_l, b1, vn0)


def _tc_final(accs, g2, dinv_b, b2):
    R = 1000

    def body(acc_ref, g_ref, dv_ref, b_ref, o_ref):
        o_ref[...] = dv_ref[...] * (acc_ref[0] + acc_ref[1] - g_ref[...]) + b_ref[...]

    return pl.pallas_call(
        body,
        grid=(N // R,),
        in_specs=[
            pl.BlockSpec((NC, R, F), lambda i: (0, i, 0)),
            pl.BlockSpec((R, F), lambda i: (i, 0)),
            pl.BlockSpec((R, F), lambda i: (i, 0)),
            pl.BlockSpec((1, F), lambda i: (0, 0)),
        ],
        out_specs=pl.BlockSpec((R, F), lambda i: (i, 0)),
        out_shape=jax.ShapeDtypeStruct((N, F), jnp.float32),
    )(accs, g2, dinv_b, b2)


def kernel(x, edge_index, h_blocks, h_levels, h_num,
           W_init, b_init, W_l, b_l, vn_emb, mW1, mb1, mW2, mb2):
    E = edge_index.shape[1]
    ept = E // NW
    nch = -(-ept // CH)
    nch = -(-nch // NB) * NB
    pad = nch * CH - ept

    src = edge_index[0].reshape(NW, ept)
    dst = edge_index[1].reshape(NW, ept)
    spread = jnp.arange(pad, dtype=jnp.int32)
    pad_src = jnp.broadcast_to(spread * 79 % N, (NW, pad))
    pad_dst = jnp.broadcast_to(N + spread % (NR - N), (NW, pad))
    srcp = jnp.concatenate([src, pad_src], axis=1).reshape(NW, nch, CH)
    dstp = jnp.concatenate([dst, pad_dst], axis=1).reshape(NW, nch, CH)
    xp = jnp.pad(x, ((0, NR - N), (0, 0)))
    zeros_deg = jnp.zeros((NR, DEGW), jnp.float32)
    ones_deg = jnp.ones((CH, DEGW), jnp.float32)

    degcnt = _deg_count(dstp, zeros_deg, ones_deg)
    g1, dinv_b = _tc_prep(degcnt, xp, W_init)
    accs1 = _aggregate(srcp, dstp, g1)
    g2 = _tc_mid(accs1, g1, dinv_b, W_l, b_init.reshape(1, F), vn_emb)
    accs2 = _aggregate(srcp, dstp, g2)
    return _tc_final(accs2, g2, dinv_b, b_l.reshape(1, F))


# speedup vs baseline: 2.5620x; 1.0018x over previous
import functools

import jax
import jax.numpy as jnp
from jax import lax
from jax.experimental import pallas as pl
from jax.experimental.pallas import tpu as pltpu
from jax.experimental.pallas import tpu_sc as plsc

N = 10000
NR = 10112
F = 128
NC = 2
NS = 16
NW = NC * NS
DEGW = 128
CH = 128
NB = 2


def _sc_mesh():
    return plsc.VectorSubcoreMesh(core_axis_name="c", subcore_axis_name="s")


def _deg_count(dstp, zeros_deg, ones_deg):
    nch = dstp.shape[1]
    rz = NR // NS

    @functools.partial(
        pl.kernel,
        out_type=jax.ShapeDtypeStruct((NC, NR, DEGW), jnp.float32),
        mesh=_sc_mesh(),
        scratch_types=[
            pltpu.VMEM((nch, CH), jnp.int32),
            pltpu.VMEM((CH, DEGW), jnp.float32),
            pltpu.VMEM_SHARED((NR, DEGW), jnp.float32),
            pltpu.SemaphoreType.DMA((8,)),
        ],
    )
    def k(dstp_hbm, zeros_hbm, ones_hbm, out_hbm, dst_v, ones_v, acc_sh, sem):
        c = lax.axis_index("c")
        s = lax.axis_index("s")
        wid = c * NS + s
        pltpu.sync_copy(dstp_hbm.at[wid], dst_v)
        pltpu.sync_copy(ones_hbm, ones_v)
        pltpu.sync_copy(zeros_hbm, acc_sh.at[pl.ds(s * rz, rz)])
        plsc.subcore_barrier()

        ND = 8
        for b in range(ND):
            pltpu.async_copy(ones_v, acc_sh.at[dst_v.at[b]], sem.at[b], add=True)

        def body(t, carry):
            for b in range(ND):
                j = t * ND + b
                pltpu.make_async_copy(ones_v, acc_sh.at[dst_v.at[j]], sem.at[b]).wait()

                @pl.when(j + ND < nch)
                def _():
                    pltpu.async_copy(
                        ones_v, acc_sh.at[dst_v.at[j + ND]], sem.at[b], add=True)
            return carry

        lax.fori_loop(0, nch // ND, body, 0)
        plsc.subcore_barrier()
        pltpu.sync_copy(acc_sh.at[pl.ds(s * rz, rz)], out_hbm.at[c, pl.ds(s * rz, rz)])

    return k(dstp, zeros_deg, ones_deg)


def _aggregate(srcp, dstp, g):
    nch = srcp.shape[1]
    hch = nch // 2
    ri = NR // NS

    @functools.partial(
        pl.kernel,
        out_type=jax.ShapeDtypeStruct((NC, NR, F), jnp.float32),
        mesh=_sc_mesh(),
        scratch_types=[
            pltpu.VMEM((hch, CH), jnp.int32),
            pltpu.VMEM((hch, CH), jnp.int32),
            pltpu.VMEM((NB, CH, F), jnp.float32),
            pltpu.VMEM_SHARED((NR, F), jnp.float32),
            pltpu.SemaphoreType.DMA((NB,)),
            pltpu.SemaphoreType.DMA((NB,)),
        ],
    )
    def k(srcp_hbm, dstp_hbm, g_hbm, out_hbm, src_v, dst_v, bufs, acc_sh, sem_g, sem_s):
        c = lax.axis_index("c")
        s = lax.axis_index("s")
        wid = c * NS + s
        pltpu.sync_copy(g_hbm.at[pl.ds(s * ri, ri)], acc_sh.at[pl.ds(s * ri, ri)])
        plsc.subcore_barrier()

        def grp(t, carry):
            for b in range(NB):
                j = t * NB + b
                pltpu.make_async_copy(
                    g_hbm.at[src_v.at[j]], bufs.at[b], sem_g.at[b]).wait()
                pltpu.async_copy(
                    bufs.at[b], acc_sh.at[dst_v.at[j]], sem_s.at[b], add=True).wait()

                @pl.when(j + NB < hch)
                def _():
                    pltpu.async_copy(
                        g_hbm.at[src_v.at[j + NB]], bufs.at[b], sem_g.at[b])
            return carry

        for h in range(2):
            pltpu.sync_copy(srcp_hbm.at[wid, pl.ds(h * hch, hch)], src_v)
            pltpu.sync_copy(dstp_hbm.at[wid, pl.ds(h * hch, hch)], dst_v)
            for b in range(NB):
                pltpu.async_copy(g_hbm.at[src_v.at[b]], bufs.at[b], sem_g.at[b])
            lax.fori_loop(0, hch // NB, grp, 0)
        plsc.subcore_barrier()
        pltpu.sync_copy(acc_sh.at[pl.ds(s * ri, ri)], out_hbm.at[c, pl.ds(s * ri, ri)])

    return k(srcp, dstp, g)


_TC_R = 1264


def _tc_prep(degcnt, xp, W):
    def body(deg_ref, x_ref, w_ref, g_ref, dinv_ref):
        dd = deg_ref[0, :, :1] + deg_ref[1, :, :1] + 1.0
        dinv = lax.rsqrt(dd)
        h = jnp.dot(x_ref[...], w_ref[...], preferred_element_type=jnp.float32)
        g_ref[...] = dinv * h
        dinv_ref[...] = jnp.broadcast_to(dinv, (_TC_R, F))

    return pl.pallas_call(
        body,
        grid=(NR // _TC_R,),
        in_specs=[
            pl.BlockSpec((NC, _TC_R, DEGW), lambda i: (0, i, 0)),
            pl.BlockSpec((_TC_R, F), lambda i: (i, 0)),
            pl.BlockSpec((F, F), lambda i: (0, 0)),
        ],
        out_specs=[
            pl.BlockSpec((_TC_R, F), lambda i: (i, 0)),
            pl.BlockSpec((_TC_R, F), lambda i: (i, 0)),
        ],
        out_shape=[
            jax.ShapeDtypeStruct((NR, F), jnp.float32),
            jax.ShapeDtypeStruct((NR, F), jnp.float32),
        ],
    )(degcnt, xp, W)


def _tc_mid(accs, g1, dinv_b, W_l, b1, vn0):
    def body(acc_ref, g_ref, dv_ref, w_ref, b_ref, vn_ref, g2_ref):
        out1 = dv_ref[...] * (acc_ref[0] + acc_ref[1] - g_ref[...])
        out1 = out1 + b_ref[...] + vn_ref[...]
        h2 = jnp.dot(out1, w_ref[...], preferred_element_type=jnp.float32)
        g2_ref[...] = dv_ref[...] * h2

    return pl.pallas_call(
        body,
        grid=(NR // _TC_R,),
        in_specs=[
            pl.BlockSpec((NC, _TC_R, F), lambda i: (0, i, 0)),
            pl.BlockSpec((_TC_R, F), lambda i: (i, 0)),
            pl.BlockSpec((_TC_R, F), lambda i: (i, 0)),
            pl.BlockSpec((F, F), lambda i: (0, 0)),
            pl.BlockSpec((1, F), lambda i: (0, 0)),
            pl.BlockSpec((1, F), lambda i: (0, 0)),
        ],
        out_specs=pl.BlockSpec((_TC_R, F), lambda i: (i, 0)),
        out_shape=jax.ShapeDtypeStruct((NR, F), jnp.float32),
    )(accs, g1, dinv_b, W_l, b1, vn0)


def _tc_final(accs, g2, dinv_b, b2):
    R = 1000

    def body(acc_ref, g_ref, dv_ref, b_ref, o_ref):
        o_ref[...] = dv_ref[...] * (acc_ref[0] + acc_ref[1] - g_ref[...]) + b_ref[...]

    return pl.pallas_call(
        body,
        grid=(N // R,),
        in_specs=[
            pl.BlockSpec((NC, R, F), lambda i: (0, i, 0)),
            pl.BlockSpec((R, F), lambda i: (i, 0)),
            pl.BlockSpec((R, F), lambda i: (i, 0)),
            pl.BlockSpec((1, F), lambda i: (0, 0)),
        ],
        out_specs=pl.BlockSpec((R, F), lambda i: (i, 0)),
        out_shape=jax.ShapeDtypeStruct((N, F), jnp.float32),
    )(accs, g2, dinv_b, b2)


def kernel(x, edge_index, h_blocks, h_levels, h_num,
           W_init, b_init, W_l, b_l, vn_emb, mW1, mb1, mW2, mb2):
    E = edge_index.shape[1]
    ept = E // NW
    nch = -(-ept // CH)
    nch = -(-nch // NB) * NB
    pad = nch * CH - ept

    src = edge_index[0].reshape(NW, ept)
    dst = edge_index[1].reshape(NW, ept)
    spread = jnp.arange(pad, dtype=jnp.int32)
    pad_src = jnp.broadcast_to(spread * 79 % N, (NW, pad))
    pad_dst = jnp.broadcast_to(N + spread % (NR - N), (NW, pad))
    srcp = jnp.concatenate([src, pad_src], axis=1).reshape(NW, nch, CH)
    dstp = jnp.concatenate([dst, pad_dst], axis=1).reshape(NW, nch, CH)
    xp = jnp.pad(x, ((0, NR - N), (0, 0)))
    zeros_deg = jnp.zeros((NR // NS, DEGW), jnp.float32)
    ones_deg = jnp.ones((CH, DEGW), jnp.float32)

    degcnt = _deg_count(dstp, zeros_deg, ones_deg)
    g1, dinv_b = _tc_prep(degcnt, xp, W_init)
    accs1 = _aggregate(srcp, dstp, g1)
    g2 = _tc_mid(accs1, g1, dinv_b, W_l, b_init.reshape(1, F), vn_emb)
    accs2 = _aggregate(srcp, dstp, g2)
    return _tc_final(accs2, g2, dinv_b, b_l.reshape(1, F))
